# Initial kernel scaffold; baseline (speedup 1.0000x reference)
import jax
import jax.numpy as jnp
from jax.experimental import pallas as pl


def kernel(x, W0, g0, b0, W1, g1, b1, Wf, gf, bf):
    raise NotImplementedError("write your pallas kernel here")


# speedup vs baseline: 2.0651x; 2.0651x over previous
import functools

import jax
import jax.numpy as jnp
from jax.experimental import pallas as pl

_K = 20
_NEG = -3.0e38
_EPS = 1e-5
_INTERPRET = False


def _knn_body(featT_ref, feat_ref, idx_ref, *, k, n, mode):
    ht = feat_ref[0]
    hT = featT_ref[0]
    xx_j = jnp.sum(hT * hT, axis=0, keepdims=True)
    xx_i = jnp.sum(ht * ht, axis=1, keepdims=True)
    if mode == "bf16":
        p = jnp.dot(ht.astype(jnp.bfloat16), hT.astype(jnp.bfloat16),
                    preferred_element_type=jnp.float32)
    else:
        prec = (jax.lax.Precision.HIGHEST if mode == "highest"
                else jax.lax.Precision.DEFAULT)
        p = jnp.dot(ht, hT, preferred_element_type=jnp.float32, precision=prec)
    d = 2.0 * p - xx_i - xx_j
    tn = ht.shape[0]
    iota = jax.lax.broadcasted_iota(jnp.int32, (tn, n), 1)
    cols = []
    for _ in range(k):
        mx = jnp.max(d, axis=1, keepdims=True)
        j = jnp.min(jnp.where(d == mx, iota, n), axis=1, keepdims=True)
        cols.append(j)
        d = jnp.where(iota == j, _NEG, d)
    idx_ref[0] = jnp.concatenate(cols, axis=1)


def _knn(feat, featT, k, tn, mode="bf16"):
    b, n, c = feat.shape
    grid = (b, n // tn)
    return pl.pallas_call(
        functools.partial(_knn_body, k=k, n=n, mode=mode),
        grid=grid,
        in_specs=[
            pl.BlockSpec((1, c, n), lambda bi, ti: (bi, 0, 0)),
            pl.BlockSpec((1, tn, c), lambda bi, ti: (bi, ti, 0)),
        ],
        out_specs=pl.BlockSpec((1, tn, k), lambda bi, ti: (bi, ti, 0)),
        out_shape=jax.ShapeDtypeStruct((b, n, k), jnp.int32),
        interpret=_INTERPRET,
    )(featT, feat)


def _edge_body(feat_ref, ftile_ref, idx_ref, wt_ref,
               m_ref, mn_ref, st_ref, *, k, n):
    featb = feat_ref[0]
    xi = ftile_ref[0]
    wt = wt_ref[...].astype(jnp.bfloat16)
    idx = idx_ref[0]
    tn = xi.shape[0]
    iota = jax.lax.broadcasted_iota(jnp.int32, (tn, n), 1)
    m = None
    for kk in range(k):
        col = idx[:, kk][:, None]
        onehot = jnp.where(iota == col, 1.0, 0.0)
        xj = jnp.dot(onehot, featb, preferred_element_type=jnp.float32,
                     precision=jax.lax.Precision.HIGHEST)
        e = jnp.concatenate([xj - xi, xi], axis=1).astype(jnp.bfloat16)
        g = jnp.dot(e, wt, preferred_element_type=jnp.float32)
        if m is None:
            m, mn, su, squ = g, g, g, g * g
        else:
            m = jnp.maximum(m, g)
            mn = jnp.minimum(mn, g)
            su = su + g
            squ = squ + g * g
    m_ref[0] = m
    mn_ref[0] = mn
    s_part = jnp.sum(su, axis=0, keepdims=True)
    ss_part = jnp.sum(squ, axis=0, keepdims=True)
    f = s_part.shape[1]
    upd = jnp.concatenate([s_part, ss_part, jnp.zeros((6, f), jnp.float32)], axis=0)

    @pl.when((pl.program_id(0) == 0) & (pl.program_id(1) == 0))
    def _():
        st_ref[...] = jnp.zeros_like(st_ref)

    st_ref[...] += upd


def _edge(feat, idx, wt, tn):
    b, n, c = feat.shape
    k = idx.shape[2]
    f = wt.shape[1]
    grid = (b, n // tn)
    return pl.pallas_call(
        functools.partial(_edge_body, k=k, n=n),
        grid=grid,
        in_specs=[
            pl.BlockSpec((1, n, c), lambda bi, ti: (bi, 0, 0)),
            pl.BlockSpec((1, tn, c), lambda bi, ti: (bi, ti, 0)),
            pl.BlockSpec((1, tn, k), lambda bi, ti: (bi, ti, 0)),
            pl.BlockSpec((2 * c, f), lambda bi, ti: (0, 0)),
        ],
        out_specs=[
            pl.BlockSpec((1, tn, f), lambda bi, ti: (bi, ti, 0)),
            pl.BlockSpec((1, tn, f), lambda bi, ti: (bi, ti, 0)),
            pl.BlockSpec((8, f), lambda bi, ti: (0, 0)),
        ],
        out_shape=[
            jax.ShapeDtypeStruct((b, n, f), jnp.float32),
            jax.ShapeDtypeStruct((b, n, f), jnp.float32),
            jax.ShapeDtypeStruct((8, f), jnp.float32),
        ],
        interpret=_INTERPRET,
    )(feat, feat, idx, wt)


def _finalize_body(m_ref, mn_ref, sc_ref, sh_ref, o_ref):
    sc = sc_ref[...]
    y = jnp.where(sc > 0, m_ref[0], mn_ref[0]) * sc + sh_ref[...]
    o_ref[0] = jnp.where(y > 0, y, 0.2 * y)


def _finalize(m, mn, scale, shift):
    b, n, f = m.shape
    return pl.pallas_call(
        _finalize_body,
        grid=(b,),
        in_specs=[
            pl.BlockSpec((1, n, f), lambda bi: (bi, 0, 0)),
            pl.BlockSpec((1, n, f), lambda bi: (bi, 0, 0)),
            pl.BlockSpec((1, f), lambda bi: (0, 0)),
            pl.BlockSpec((1, f), lambda bi: (0, 0)),
        ],
        out_specs=pl.BlockSpec((1, n, f), lambda bi: (bi, 0, 0)),
        out_shape=jax.ShapeDtypeStruct((b, n, f), jnp.float32),
        interpret=_INTERPRET,
    )(m, mn, scale, shift)


def _fuse_body(h1_ref, m_ref, mn_ref, sc_ref, sh_ref, wf_ref, y_ref, st_ref):
    sc = sc_ref[...]
    t = jnp.where(sc > 0, m_ref[0], mn_ref[0]) * sc + sh_ref[...]
    h2 = jnp.where(t > 0, t, 0.2 * t)
    cat = jnp.concatenate([h1_ref[0], h2], axis=1).astype(jnp.bfloat16)
    y = jnp.dot(cat, wf_ref[...].astype(jnp.bfloat16),
                preferred_element_type=jnp.float32)
    y_ref[0] = y
    s_part = jnp.sum(y, axis=0, keepdims=True)
    ss_part = jnp.sum(y * y, axis=0, keepdims=True)
    e = s_part.shape[1]
    upd = jnp.concatenate([s_part, ss_part, jnp.zeros((6, e), jnp.float32)], axis=0)

    @pl.when((pl.program_id(0) == 0) & (pl.program_id(1) == 0))
    def _():
        st_ref[...] = jnp.zeros_like(st_ref)

    st_ref[...] += upd


def _fuse(h1, m2, mn2, scale2, shift2, wfT, tn):
    b, n, f = h1.shape
    e = wfT.shape[1]
    grid = (b, n // tn)
    return pl.pallas_call(
        _fuse_body,
        grid=grid,
        in_specs=[
            pl.BlockSpec((1, tn, f), lambda bi, ti: (bi, ti, 0)),
            pl.BlockSpec((1, tn, f), lambda bi, ti: (bi, ti, 0)),
            pl.BlockSpec((1, tn, f), lambda bi, ti: (bi, ti, 0)),
            pl.BlockSpec((1, f), lambda bi, ti: (0, 0)),
            pl.BlockSpec((1, f), lambda bi, ti: (0, 0)),
            pl.BlockSpec((2 * f, e), lambda bi, ti: (0, 0)),
        ],
        out_specs=[
            pl.BlockSpec((1, tn, e), lambda bi, ti: (bi, ti, 0)),
            pl.BlockSpec((8, e), lambda bi, ti: (0, 0)),
        ],
        out_shape=[
            jax.ShapeDtypeStruct((b, n, e), jnp.float32),
            jax.ShapeDtypeStruct((8, e), jnp.float32),
        ],
        interpret=_INTERPRET,
    )(h1, m2, mn2, scale2, shift2, wfT)


def _bn_params(st, gamma, beta, count):
    s = st[0]
    ss = st[1]
    mean = s / count
    var = ss / count - mean * mean
    scale = gamma * jax.lax.rsqrt(var + _EPS)
    shift = beta - mean * scale
    return scale[None, :], shift[None, :]


def kernel(x, W0, g0, b0, W1, g1, b1, Wf, gf, bf):
    b, n, c0 = x.shape
    f0 = W0.shape[0]
    k = _K

    cp = 8
    xp = jnp.concatenate([x, jnp.zeros((b, n, cp - c0), x.dtype)], axis=-1)
    xpT = jnp.transpose(xp, (0, 2, 1))

    zpad = jnp.zeros((cp - c0, f0), jnp.float32)
    wt0 = jnp.concatenate([jnp.transpose(W0[:, :c0]), zpad,
                           jnp.transpose(W0[:, c0:]), zpad], axis=0)

    tn = min(256, n)
    idx0 = _knn(xp, xpT, k, tn=tn)
    m1, mn1, st1 = _edge(xp, idx0, wt0, tn=tn)
    sc1, sh1 = _bn_params(st1, g0, b0, float(b * n * k))
    h1 = _finalize(m1, mn1, sc1, sh1)

    h1T = jnp.transpose(h1, (0, 2, 1))
    wt1 = jnp.transpose(W1)

    idx1 = _knn(h1, h1T, k, tn=tn)
    m2, mn2, st2 = _edge(h1, idx1, wt1, tn=tn)
    sc2, sh2 = _bn_params(st2, g1, b1, float(b * n * k))

    yT, stf = _fuse(h1, m2, mn2, sc2, sh2, jnp.transpose(Wf), tn=min(512, n))
    scf, shf = _bn_params(stf, gf, bf, float(b * n))
    return _finalize(yT, yT, scf, shf)
